# baseline (speedup 1.0000x reference)
import functools
import math

import jax
import jax.numpy as jnp
from jax import lax
from jax.experimental import pallas as pl
from jax.experimental.pallas import tpu as pltpu
from jax.experimental.pallas import tpu_sc as plsc

D_MODEL = 768
_SCALE = math.sqrt(D_MODEL)

_info = plsc.get_sparse_core_info()
_NC = _info.num_cores
_NS = _info.num_subcores
_L = _info.num_lanes
_NW = _NC * _NS

_CHUNK = 16
_NB = 4


def _make_kernel(B: int):
    assert B % (_NW * _CHUNK) == 0
    b_per_w = B // _NW
    n_chunks = b_per_w // _CHUNK
    assert n_chunks % _NB == 0 and n_chunks >= 2 * _NB
    n_vecs = D_MODEL // _L

    mesh = plsc.VectorSubcoreMesh(core_axis_name="c", subcore_axis_name="s")

    @functools.partial(
        pl.kernel,
        mesh=mesh,
        out_type=jax.ShapeDtypeStruct((B, D_MODEL), jnp.float32),
        scratch_types=(
            [pltpu.VMEM((n_chunks, _CHUNK), jnp.int32)]
            + [pltpu.VMEM((_CHUNK, D_MODEL), jnp.float32)] * (2 * _NB)
            + [pltpu.SemaphoreType.DMA] * (2 * _NB)
        ),
    )
    def emb_kernel(table_hbm, x_hbm, out_hbm, idx_v, *bufs_and_sems):
        gbufs = bufs_and_sems[:_NB]
        obufs = bufs_and_sems[_NB:2 * _NB]
        gsems = bufs_and_sems[2 * _NB:3 * _NB]
        osems = bufs_and_sems[3 * _NB:4 * _NB]

        wid = lax.axis_index("s") * _NC + lax.axis_index("c")
        base = wid * b_per_w

        pltpu.sync_copy(x_hbm.at[wid], idx_v)

        def issue_gather(c, b):
            pltpu.async_copy(table_hbm.at[idx_v.at[c]], gbufs[b], gsems[b])

        def wait_gather(b):
            pltpu.make_async_copy(
                table_hbm.at[idx_v.at[0]], gbufs[b], gsems[b]).wait()

        def issue_write(c, b):
            pltpu.async_copy(
                obufs[b], out_hbm.at[pl.ds(base + c * _CHUNK, _CHUNK)],
                osems[b])

        def wait_write(b):
            pltpu.make_async_copy(
                obufs[b], out_hbm.at[pl.ds(base, _CHUNK)], osems[b]).wait()

        def scale(b):
            src = gbufs[b]
            dst = obufs[b]
            def row_body(r, carry):
                for j in range(n_vecs):
                    sl = (r, pl.ds(j * _L, _L))
                    dst[sl] = src[sl] * _SCALE
                return carry
            lax.fori_loop(0, _CHUNK, row_body, 0)

        for b in range(_NB):
            issue_gather(b, b)
        for b in range(_NB):
            wait_gather(b)
            scale(b)
            issue_gather(b + _NB, b)
            issue_write(b, b)

        def loop_body(i, carry):
            g = _NB + _NB * i
            for b in range(_NB):
                c = g + b
                wait_gather(b)
                wait_write(b)
                scale(b)
                issue_gather(c + _NB, b)
                issue_write(c, b)
            return carry
        lax.fori_loop(0, (n_chunks - 2 * _NB) // _NB, loop_body, 0)

        for b in range(_NB):
            c = n_chunks - _NB + b
            wait_gather(b)
            wait_write(b)
            scale(b)
            issue_write(c, b)
        for b in range(_NB):
            wait_write(b)

    return emb_kernel


def kernel(table, x):
    B = x.size
    x_blocked = x.reshape(_NW, B // _NW // _CHUNK, _CHUNK)
    out = _make_kernel(B)(table, x_blocked)
    return out.reshape(x.shape + (D_MODEL,))
